# baseline (speedup 1.0000x reference)
import functools

import jax
import jax.numpy as jnp
from jax import lax
from jax.experimental import pallas as pl
from jax.experimental.pallas import tpu as pltpu
from jax.experimental.pallas import tpu_sc as plsc

N = 10000
E = 160000
DIN = 128
DE = 16
D = 32
EH = 128
STEPS = 6
S2S = 6
L = 3
RH = 128
NT = 1

F32 = jnp.float32

NC = 2
NS = 16
NW = NC * NS
CH = 125
NCHUNK = E // CH
WCH = NCHUNK // NW
GRP = 8
NGRP = WCH // GRP


def _sigmoid(x):
    return 1.0 / (1.0 + jnp.exp(-x))


def _dot(a, b):
    return jnp.dot(a.astype(jnp.bfloat16), b.astype(jnp.bfloat16),
                   preferred_element_type=F32)


def _b(x):
    u = lax.bitcast_convert_type(x, jnp.uint32)
    bias = jnp.uint32(0x7FFF) + ((u >> 16) & jnp.uint32(1))
    u = (u + bias) & jnp.uint32(0xFFFF0000)
    return lax.bitcast_convert_type(u, F32)


def _proj_body(nf_ref, wp_ref, bp_ref, out_ref):
    out_ref[...] = jnp.maximum(
        _dot(nf_ref[...], wp_ref[...]) + bp_ref[...], 0.0)


def _proj(nf, wp, bp):
    return pl.pallas_call(
        _proj_body,
        out_shape=jax.ShapeDtypeStruct((N, D), F32),
    )(nf, wp, bp)


BE = 640


def _edgenet_body(ef_ref, w1_ref, b1_ref, w2_ref, b2_ref, ew_ref):
    z = jnp.maximum(
        _dot(ef_ref[...], w1_ref[...]) + b1_ref[...], 0.0)
    ew_ref[...] = (_dot(_b(z), w2_ref[...])
                   + b2_ref[...]).astype(ew_ref.dtype)


def _edgenet(ef, w1, b1, w2, b2, dtype):
    return pl.pallas_call(
        _edgenet_body,
        grid=(E // BE,),
        in_specs=[
            pl.BlockSpec((BE, DE), lambda i: (i, 0)),
            pl.BlockSpec((DE, EH), lambda i: (0, 0)),
            pl.BlockSpec((1, EH), lambda i: (0, 0)),
            pl.BlockSpec((EH, D * D), lambda i: (0, 0)),
            pl.BlockSpec((1, D * D), lambda i: (0, 0)),
        ],
        out_specs=pl.BlockSpec((BE, D * D), lambda i: (i, 0)),
        out_shape=jax.ShapeDtypeStruct((E, D * D), dtype),
    )(ef, w1, b1, w2, b2)


def _msg_body(hg_ref, ew_ref, m_ref):
    hg = _b(hg_ref[...])
    ew = ew_ref[...].astype(F32)
    acc = hg[:, 0:1] * ew[:, 0:D]
    for i in range(1, D):
        acc += hg[:, i:i + 1] * ew[:, i * D:(i + 1) * D]
    m_ref[...] = acc


def _msg(hg, ew):
    return pl.pallas_call(
        _msg_body,
        grid=(E // BE,),
        in_specs=[
            pl.BlockSpec((BE, D), lambda i: (i, 0)),
            pl.BlockSpec((BE, D * D), lambda i: (i, 0)),
        ],
        out_specs=pl.BlockSpec((BE, D), lambda i: (i, 0)),
        out_shape=jax.ShapeDtypeStruct((E, D), F32),
    )(hg, ew)


def _gru_body(a0_ref, a1_ref, bc_ref, h_ref, wih_ref, whh_ref, bih_ref,
              bhh_ref, out_ref):
    a = jnp.maximum(a0_ref[...] + a1_ref[...] + bc_ref[...], 0.0)
    h = h_ref[...]
    gi = _dot(_b(a), wih_ref[...]) + bih_ref[...]
    gh = _dot(h, whh_ref[...]) + bhh_ref[...]
    r = _sigmoid(gi[:, :D] + gh[:, :D])
    z = _sigmoid(gi[:, D:2 * D] + gh[:, D:2 * D])
    n = jnp.tanh(gi[:, 2 * D:] + r * gh[:, 2 * D:])
    out_ref[...] = (1.0 - z) * n + z * h


def _gru(a0, a1, bc, h, wih_t, whh_t, bih, bhh):
    return pl.pallas_call(
        _gru_body,
        out_shape=jax.ShapeDtypeStruct((N, D), F32),
    )(a0, a1, bc, h, wih_t, whh_t, bih, bhh)


def _s2s_body(h_ref,
              w0a_ref, w0b_ref, u0_ref, b0_ref,
              w1_ref, u1_ref, b1_ref,
              w2_ref, u2_ref, b2_ref,
              wr1a_ref, wr1b_ref, br1_ref, wr2_ref, br2_ref,
              out_ref):
    h = h_ref[...]
    q = jnp.zeros((1, D), F32)
    rd = jnp.zeros((1, D), F32)
    hs = [jnp.zeros((1, D), F32) for _ in range(L)]
    cs = [jnp.zeros((1, D), F32) for _ in range(L)]
    ws = [(None, u0_ref, b0_ref), (w1_ref, u1_ref, b1_ref),
          (w2_ref, u2_ref, b2_ref)]
    hb = _b(h)
    for _ in range(S2S):
        x = None
        for l in range(L):
            w, u, b = ws[l]
            if l == 0:
                g = _dot(_b(q), w0a_ref[...]) + _dot(rd, w0b_ref[...])
            else:
                g = _dot(x, w[...])
            g = g + _dot(hs[l], u[...]) + b[...]
            i = _sigmoid(g[:, :D])
            f = _sigmoid(g[:, D:2 * D])
            gg = jnp.tanh(g[:, 2 * D:3 * D])
            o = _sigmoid(g[:, 3 * D:])
            cs[l] = f * cs[l] + i * gg
            hs[l] = o * jnp.tanh(cs[l])
            x = hs[l]
        q = x
        e = jnp.sum(hb * _b(q), axis=1, keepdims=True)
        e = e - jnp.max(e, axis=0, keepdims=True)
        ex = jnp.exp(e)
        alpha = ex / jnp.sum(ex, axis=0, keepdims=True)
        rd = jnp.sum(_b(alpha) * hb, axis=0, keepdims=True)
    hid = jnp.maximum(
        _dot(q, wr1a_ref[...]) + _dot(rd, wr1b_ref[...])
        + br1_ref[...], 0.0)
    out_ref[...] = _dot(hid, wr2_ref[...]) + br2_ref[...]


def _s2s(h, args):
    return pl.pallas_call(
        _s2s_body,
        out_shape=jax.ShapeDtypeStruct((1, NT), F32),
    )(h, *args)


@functools.cache
def _mesh():
    return plsc.VectorSubcoreMesh(core_axis_name="c", subcore_axis_name="s",
                                  num_cores=NC, num_subcores=NS)


def _sc_gather_body(h_hbm, src_hbm, out_hbm, idx_v, rows_v, sem):
    cid = lax.axis_index("c")
    sid = lax.axis_index("s")
    wid = sid * NC + cid
    cstart = wid * WCH
    pltpu.sync_copy(src_hbm.at[pl.ds(cstart, WCH)], idx_v)

    @pl.loop(0, NGRP)
    def _grp(g):
        descs = []
        for j in range(GRP):
            descs.append(pltpu.async_copy(
                h_hbm.at[idx_v.at[g * GRP + j, 0]], rows_v.at[j], sem))
        for d in descs:
            d.wait()
        pltpu.sync_copy(rows_v, out_hbm.at[pl.ds(cstart + g * GRP, GRP)])


@functools.cache
def _sc_gather_kernel():
    return pl.kernel(
        _sc_gather_body,
        out_type=jax.ShapeDtypeStruct((NCHUNK, CH, D), F32),
        mesh=_mesh(),
        scratch_types=[
            pltpu.VMEM((WCH, 1, CH), jnp.int32),
            pltpu.VMEM((GRP, CH, D), F32),
            pltpu.SemaphoreType.DMA,
        ],
        compiler_params=pltpu.CompilerParams(use_tc_tiling_on_sc=False),
    )


def _sc_gather(h, src3):
    return _sc_gather_kernel()(h, src3).reshape(E, D)


NEXP = 10


def _sc_scatter_body(m_hbm, dst_hbm, zero_hbm, out_hbm, shared, idx_v,
                     rows_v, sem):
    cid = lax.axis_index("c")
    sid = lax.axis_index("s")
    wid = sid * NC + cid
    cstart = wid * WCH

    @pl.when(sid == 0)
    def _zero():
        pltpu.sync_copy(zero_hbm, shared)

    plsc.subcore_barrier()
    pltpu.sync_copy(dst_hbm.at[pl.ds(cstart, WCH)], idx_v)

    @pl.loop(0, NGRP)
    def _grp(g):
        pltpu.sync_copy(m_hbm.at[pl.ds(cstart + g * GRP, GRP)], rows_v)
        for j in range(GRP):
            pltpu.sync_copy(rows_v.at[j],
                            shared.at[idx_v.at[g * GRP + j, 0]], add=True)

    plsc.subcore_barrier()

    @pl.when(sid < NEXP)
    def _export():
        rpt = N // NEXP
        pltpu.sync_copy(shared.at[pl.ds(sid * rpt, rpt)],
                        out_hbm.at[cid].at[pl.ds(sid * rpt, rpt)])


@functools.cache
def _sc_scatter_kernel():
    return pl.kernel(
        _sc_scatter_body,
        out_type=jax.ShapeDtypeStruct((NC, N, D), F32),
        mesh=_mesh(),
        scratch_types=[
            pltpu.VMEM_SHARED((N, D), F32),
            pltpu.VMEM((WCH, 1, CH), jnp.int32),
            pltpu.VMEM((GRP, CH, D), F32),
            pltpu.SemaphoreType.DMA,
        ],
        compiler_params=pltpu.CompilerParams(use_tc_tiling_on_sc=False),
    )


def _sc_scatter(m, dst3, zero):
    return _sc_scatter_kernel()(m.reshape(NCHUNK, CH, D), dst3, zero)


def kernel(node_feats, edge_index, edge_feats, W_p, b_p, We1, be1, We2, be2,
           b_conv, gru_Wih, gru_Whh, gru_bih, gru_bhh,
           lstm_Wih0, lstm_Whh0, lstm_bih0, lstm_bhh0,
           lstm_Wih1, lstm_Whh1, lstm_bih1, lstm_bhh1,
           lstm_Wih2, lstm_Whh2, lstm_bih2, lstm_bhh2,
           Wr1, br1, Wr2, br2):
    src3 = edge_index[0].astype(jnp.int32).reshape(NCHUNK, 1, CH)
    dst3 = edge_index[1].astype(jnp.int32).reshape(NCHUNK, 1, CH)

    ew = _edgenet(edge_feats, We1, be1.reshape(1, EH), We2,
                  be2.reshape(1, D * D), jnp.bfloat16)
    h = _proj(node_feats, W_p, b_p.reshape(1, D))

    wih_t = gru_Wih.T
    whh_t = gru_Whh.T
    bih = gru_bih.reshape(1, 3 * D)
    bhh = gru_bhh.reshape(1, 3 * D)
    bc = b_conv.reshape(1, D)
    zero = jnp.zeros((N, D), F32)

    for _ in range(STEPS):
        hg = _sc_gather(h, src3)
        m = _msg(hg, ew)
        aggp = _sc_scatter(m, dst3, zero)
        h = _gru(aggp[0], aggp[1], bc, h, wih_t, whh_t, bih, bhh)

    w0t = lstm_Wih0.T
    s2s_args = (
        w0t[:D], w0t[D:], lstm_Whh0.T, lstm_bih0.reshape(1, 4 * D)
        + lstm_bhh0.reshape(1, 4 * D),
        lstm_Wih1.T, lstm_Whh1.T, lstm_bih1.reshape(1, 4 * D)
        + lstm_bhh1.reshape(1, 4 * D),
        lstm_Wih2.T, lstm_Whh2.T, lstm_bih2.reshape(1, 4 * D)
        + lstm_bhh2.reshape(1, 4 * D),
        Wr1[:D], Wr1[D:], br1.reshape(1, RH), Wr2, br2.reshape(1, NT),
    )
    return _s2s(h, s2s_args)


# speedup vs baseline: 2.3342x; 2.3342x over previous
import functools

import jax
import jax.numpy as jnp
from jax import lax
from jax.experimental import pallas as pl
from jax.experimental.pallas import tpu as pltpu
from jax.experimental.pallas import tpu_sc as plsc

N = 10000
E = 160000
DIN = 128
DE = 16
D = 32
EH = 128
STEPS = 6
S2S = 6
L = 3
RH = 128
NT = 1

F32 = jnp.float32

NC = 2
NS = 16
NW = NC * NS
CH = 125
NCHUNK = E // CH
WCH = NCHUNK // NW
GRP = 8
NGRP = WCH // GRP


def _sigmoid(x):
    return 1.0 / (1.0 + jnp.exp(-x))


def _dot(a, b):
    return jnp.dot(a.astype(jnp.bfloat16), b.astype(jnp.bfloat16),
                   preferred_element_type=F32)


def _b(x):
    u = lax.bitcast_convert_type(x, jnp.uint32)
    bias = jnp.uint32(0x7FFF) + ((u >> 16) & jnp.uint32(1))
    u = (u + bias) & jnp.uint32(0xFFFF0000)
    return lax.bitcast_convert_type(u, F32)


def _proj_body(nf_ref, wp_ref, bp_ref, out_ref):
    out_ref[...] = jnp.maximum(
        _dot(nf_ref[...], wp_ref[...]) + bp_ref[...], 0.0)


def _proj(nf, wp, bp):
    return pl.pallas_call(
        _proj_body,
        out_shape=jax.ShapeDtypeStruct((N, D), F32),
    )(nf, wp, bp)


BE = 640


def _edgenet_body(ef_ref, w1_ref, b1_ref, w2_ref, b2_ref, ew_ref):
    z = jnp.maximum(
        _dot(ef_ref[...], w1_ref[...]) + b1_ref[...], 0.0)
    ew_ref[...] = (_dot(_b(z), w2_ref[...])
                   + b2_ref[...]).astype(ew_ref.dtype)


def _edgenet(ef, w1, b1, w2, b2, dtype):
    return pl.pallas_call(
        _edgenet_body,
        grid=(E // BE,),
        in_specs=[
            pl.BlockSpec((BE, DE), lambda i: (i, 0)),
            pl.BlockSpec((DE, EH), lambda i: (0, 0)),
            pl.BlockSpec((1, EH), lambda i: (0, 0)),
            pl.BlockSpec((EH, D * D), lambda i: (0, 0)),
            pl.BlockSpec((1, D * D), lambda i: (0, 0)),
        ],
        out_specs=pl.BlockSpec((BE, D * D), lambda i: (i, 0)),
        out_shape=jax.ShapeDtypeStruct((E, D * D), dtype),
    )(ef, w1, b1, w2, b2)


def _msg_body(hg_ref, ew_ref, bmat_ref, rmat_ref, m_ref):
    hgx = jnp.dot(hg_ref[...].astype(jnp.bfloat16), bmat_ref[...],
                  preferred_element_type=F32)
    p = hgx * ew_ref[...].astype(F32)
    m_ref[...] = jnp.dot(p, rmat_ref[...], preferred_element_type=F32)


def _msg(hg, ew, bmat, rmat):
    return pl.pallas_call(
        _msg_body,
        grid=(E // BE,),
        in_specs=[
            pl.BlockSpec((BE, D), lambda i: (i, 0)),
            pl.BlockSpec((BE, D * D), lambda i: (i, 0)),
            pl.BlockSpec((D, D * D), lambda i: (0, 0)),
            pl.BlockSpec((D * D, D), lambda i: (0, 0)),
        ],
        out_specs=pl.BlockSpec((BE, D), lambda i: (i, 0)),
        out_shape=jax.ShapeDtypeStruct((E, D), F32),
    )(hg, ew, bmat, rmat)


def _gru_body(a0_ref, a1_ref, bc_ref, h_ref, wih_ref, whh_ref, bih_ref,
              bhh_ref, out_ref):
    a = jnp.maximum(a0_ref[...] + a1_ref[...] + bc_ref[...], 0.0)
    h = h_ref[...]
    gi = _dot(_b(a), wih_ref[...]) + bih_ref[...]
    gh = _dot(h, whh_ref[...]) + bhh_ref[...]
    r = _sigmoid(gi[:, :D] + gh[:, :D])
    z = _sigmoid(gi[:, D:2 * D] + gh[:, D:2 * D])
    n = jnp.tanh(gi[:, 2 * D:] + r * gh[:, 2 * D:])
    out_ref[...] = (1.0 - z) * n + z * h


def _gru(a0, a1, bc, h, wih_t, whh_t, bih, bhh):
    return pl.pallas_call(
        _gru_body,
        out_shape=jax.ShapeDtypeStruct((N, D), F32),
    )(a0, a1, bc, h, wih_t, whh_t, bih, bhh)


def _s2s_body(h_ref,
              w0a_ref, w0b_ref, u0_ref, b0_ref,
              w1_ref, u1_ref, b1_ref,
              w2_ref, u2_ref, b2_ref,
              wr1a_ref, wr1b_ref, br1_ref, wr2_ref, br2_ref,
              out_ref):
    h = h_ref[...]
    q = jnp.zeros((1, D), F32)
    rd = jnp.zeros((1, D), F32)
    hs = [jnp.zeros((1, D), F32) for _ in range(L)]
    cs = [jnp.zeros((1, D), F32) for _ in range(L)]
    ws = [(None, u0_ref, b0_ref), (w1_ref, u1_ref, b1_ref),
          (w2_ref, u2_ref, b2_ref)]
    hb = _b(h)
    for _ in range(S2S):
        x = None
        for l in range(L):
            w, u, b = ws[l]
            if l == 0:
                g = _dot(_b(q), w0a_ref[...]) + _dot(rd, w0b_ref[...])
            else:
                g = _dot(x, w[...])
            g = g + _dot(hs[l], u[...]) + b[...]
            i = _sigmoid(g[:, :D])
            f = _sigmoid(g[:, D:2 * D])
            gg = jnp.tanh(g[:, 2 * D:3 * D])
            o = _sigmoid(g[:, 3 * D:])
            cs[l] = f * cs[l] + i * gg
            hs[l] = o * jnp.tanh(cs[l])
            x = hs[l]
        q = x
        e = jnp.sum(hb * _b(q), axis=1, keepdims=True)
        e = e - jnp.max(e, axis=0, keepdims=True)
        ex = jnp.exp(e)
        alpha = ex / jnp.sum(ex, axis=0, keepdims=True)
        rd = jnp.sum(_b(alpha) * hb, axis=0, keepdims=True)
    hid = jnp.maximum(
        _dot(q, wr1a_ref[...]) + _dot(rd, wr1b_ref[...])
        + br1_ref[...], 0.0)
    out_ref[...] = _dot(hid, wr2_ref[...]) + br2_ref[...]


def _s2s(h, args):
    return pl.pallas_call(
        _s2s_body,
        out_shape=jax.ShapeDtypeStruct((1, NT), F32),
    )(h, *args)


@functools.cache
def _mesh():
    return plsc.VectorSubcoreMesh(core_axis_name="c", subcore_axis_name="s",
                                  num_cores=NC, num_subcores=NS)


def _sc_gather_body(h_hbm, src_hbm, out_hbm, idx_v, rows_v, sem):
    cid = lax.axis_index("c")
    sid = lax.axis_index("s")
    wid = sid * NC + cid
    cstart = wid * WCH
    pltpu.sync_copy(src_hbm.at[pl.ds(cstart, WCH)], idx_v)

    @pl.loop(0, NGRP)
    def _grp(g):
        descs = []
        for j in range(GRP):
            descs.append(pltpu.async_copy(
                h_hbm.at[idx_v.at[g * GRP + j, 0]], rows_v.at[j], sem))
        for d in descs:
            d.wait()
        pltpu.sync_copy(rows_v, out_hbm.at[pl.ds(cstart + g * GRP, GRP)])


@functools.cache
def _sc_gather_kernel():
    return pl.kernel(
        _sc_gather_body,
        out_type=jax.ShapeDtypeStruct((NCHUNK, CH, D), F32),
        mesh=_mesh(),
        scratch_types=[
            pltpu.VMEM((WCH, 1, CH), jnp.int32),
            pltpu.VMEM((GRP, CH, D), F32),
            pltpu.SemaphoreType.DMA,
        ],
        compiler_params=pltpu.CompilerParams(use_tc_tiling_on_sc=False),
    )


def _sc_gather(h, src3):
    return _sc_gather_kernel()(h, src3).reshape(E, D)


NEXP = 10


def _sc_scatter_body(m_hbm, dst_hbm, zero_hbm, out_hbm, shared, idx_v,
                     rows_v, sem):
    cid = lax.axis_index("c")
    sid = lax.axis_index("s")
    wid = sid * NC + cid
    cstart = wid * WCH

    @pl.when(sid == 0)
    def _zero():
        pltpu.sync_copy(zero_hbm, shared)

    plsc.subcore_barrier()
    pltpu.sync_copy(dst_hbm.at[pl.ds(cstart, WCH)], idx_v)

    @pl.loop(0, NGRP)
    def _grp(g):
        pltpu.sync_copy(m_hbm.at[pl.ds(cstart + g * GRP, GRP)], rows_v)
        for j in range(GRP):
            pltpu.sync_copy(rows_v.at[j],
                            shared.at[idx_v.at[g * GRP + j, 0]], add=True)

    plsc.subcore_barrier()

    @pl.when(sid < NEXP)
    def _export():
        rpt = N // NEXP
        pltpu.sync_copy(shared.at[pl.ds(sid * rpt, rpt)],
                        out_hbm.at[cid].at[pl.ds(sid * rpt, rpt)])


@functools.cache
def _sc_scatter_kernel():
    return pl.kernel(
        _sc_scatter_body,
        out_type=jax.ShapeDtypeStruct((NC, N, D), F32),
        mesh=_mesh(),
        scratch_types=[
            pltpu.VMEM_SHARED((N, D), F32),
            pltpu.VMEM((WCH, 1, CH), jnp.int32),
            pltpu.VMEM((GRP, CH, D), F32),
            pltpu.SemaphoreType.DMA,
        ],
        compiler_params=pltpu.CompilerParams(use_tc_tiling_on_sc=False),
    )


def _sc_scatter(m, dst3, zero):
    return _sc_scatter_kernel()(m.reshape(NCHUNK, CH, D), dst3, zero)


def kernel(node_feats, edge_index, edge_feats, W_p, b_p, We1, be1, We2, be2,
           b_conv, gru_Wih, gru_Whh, gru_bih, gru_bhh,
           lstm_Wih0, lstm_Whh0, lstm_bih0, lstm_bhh0,
           lstm_Wih1, lstm_Whh1, lstm_bih1, lstm_bhh1,
           lstm_Wih2, lstm_Whh2, lstm_bih2, lstm_bhh2,
           Wr1, br1, Wr2, br2):
    src3 = edge_index[0].astype(jnp.int32).reshape(NCHUNK, 1, CH)
    dst3 = edge_index[1].astype(jnp.int32).reshape(NCHUNK, 1, CH)

    ew = _edgenet(edge_feats, We1, be1.reshape(1, EH), We2,
                  be2.reshape(1, D * D), jnp.bfloat16)
    h = _proj(node_feats, W_p, b_p.reshape(1, D))

    wih_t = gru_Wih.T
    whh_t = gru_Whh.T
    bih = gru_bih.reshape(1, 3 * D)
    bhh = gru_bhh.reshape(1, 3 * D)
    bc = b_conv.reshape(1, D)
    zero = jnp.zeros((N, D), F32)
    bmat = jnp.kron(jnp.eye(D, dtype=jnp.bfloat16),
                    jnp.ones((1, D), jnp.bfloat16))
    rmat = jnp.tile(jnp.eye(D, dtype=F32), (D, 1))

    for _ in range(STEPS):
        hg = _sc_gather(h, src3)
        m = _msg(hg, ew, bmat, rmat)
        aggp = _sc_scatter(m, dst3, zero)
        h = _gru(aggp[0], aggp[1], bc, h, wih_t, whh_t, bih, bhh)

    w0t = lstm_Wih0.T
    s2s_args = (
        w0t[:D], w0t[D:], lstm_Whh0.T, lstm_bih0.reshape(1, 4 * D)
        + lstm_bhh0.reshape(1, 4 * D),
        lstm_Wih1.T, lstm_Whh1.T, lstm_bih1.reshape(1, 4 * D)
        + lstm_bhh1.reshape(1, 4 * D),
        lstm_Wih2.T, lstm_Whh2.T, lstm_bih2.reshape(1, 4 * D)
        + lstm_bhh2.reshape(1, 4 * D),
        Wr1[:D], Wr1[D:], br1.reshape(1, RH), Wr2, br2.reshape(1, NT),
    )
    return _s2s(h, s2s_args)


# speedup vs baseline: 2.8730x; 1.2309x over previous
import functools

import jax
import jax.numpy as jnp
from jax import lax
from jax.experimental import pallas as pl
from jax.experimental.pallas import tpu as pltpu
from jax.experimental.pallas import tpu_sc as plsc

N = 10000
E = 160000
DIN = 128
DE = 16
D = 32
EH = 128
STEPS = 6
S2S = 6
L = 3
RH = 128
NT = 1

F32 = jnp.float32

NC = 2
NS = 16
NW = NC * NS
CH = 125
NCHUNK = E // CH
WCH = NCHUNK // NW
GRP = 8
NGRP = WCH // GRP


def _sigmoid(x):
    return 1.0 / (1.0 + jnp.exp(-x))


def _dot(a, b):
    return jnp.dot(a.astype(jnp.bfloat16), b.astype(jnp.bfloat16),
                   preferred_element_type=F32)


def _b(x):
    u = lax.bitcast_convert_type(x, jnp.uint32)
    bias = jnp.uint32(0x7FFF) + ((u >> 16) & jnp.uint32(1))
    u = (u + bias) & jnp.uint32(0xFFFF0000)
    return lax.bitcast_convert_type(u, F32)


def _proj_body(nf_ref, wp_ref, bp_ref, out_ref):
    out_ref[...] = jnp.maximum(
        _dot(nf_ref[...], wp_ref[...]) + bp_ref[...], 0.0)


def _proj(nf, wp, bp):
    return pl.pallas_call(
        _proj_body,
        out_shape=jax.ShapeDtypeStruct((N, D), F32),
    )(nf, wp, bp)


BE = 1600


def _edgenet_body(ef_ref, w1_ref, b1_ref, w2_ref, b2_ref, ew_ref):
    z = jnp.maximum(
        _dot(ef_ref[...], w1_ref[...]) + b1_ref[...], 0.0)
    ew_ref[...] = (_dot(_b(z), w2_ref[...])
                   + b2_ref[...]).astype(ew_ref.dtype)


def _edgenet(ef, w1, b1, w2, b2, dtype):
    return pl.pallas_call(
        _edgenet_body,
        grid=(E // BE,),
        in_specs=[
            pl.BlockSpec((BE, DE), lambda i: (i, 0)),
            pl.BlockSpec((DE, EH), lambda i: (0, 0)),
            pl.BlockSpec((1, EH), lambda i: (0, 0)),
            pl.BlockSpec((EH, D * D), lambda i: (0, 0)),
            pl.BlockSpec((1, D * D), lambda i: (0, 0)),
        ],
        out_specs=pl.BlockSpec((BE, D * D), lambda i: (i, 0)),
        out_shape=jax.ShapeDtypeStruct((E, D * D), dtype),
    )(ef, w1, b1, w2, b2)


def _msg_body(hg_ref, ew_ref, bmat_ref, rmat_ref, m_ref):
    hgx = jnp.dot(hg_ref[...].astype(jnp.bfloat16), bmat_ref[...],
                  preferred_element_type=F32)
    p = hgx * ew_ref[...].astype(F32)
    m_ref[...] = jnp.dot(p, rmat_ref[...], preferred_element_type=F32)


def _msg(hg, ew, bmat, rmat):
    return pl.pallas_call(
        _msg_body,
        grid=(E // BE,),
        in_specs=[
            pl.BlockSpec((BE, D), lambda i: (i, 0)),
            pl.BlockSpec((BE, D * D), lambda i: (i, 0)),
            pl.BlockSpec((D, D * D), lambda i: (0, 0)),
            pl.BlockSpec((D * D, D), lambda i: (0, 0)),
        ],
        out_specs=pl.BlockSpec((BE, D), lambda i: (i, 0)),
        out_shape=jax.ShapeDtypeStruct((E, D), F32),
    )(hg, ew, bmat, rmat)


def _gru_body(a0_ref, a1_ref, bc_ref, h_ref, wih_ref, whh_ref, bih_ref,
              bhh_ref, out_ref):
    a = jnp.maximum(a0_ref[...] + a1_ref[...] + bc_ref[...], 0.0)
    h = h_ref[...]
    gi = _dot(_b(a), wih_ref[...]) + bih_ref[...]
    gh = _dot(h, whh_ref[...]) + bhh_ref[...]
    r = _sigmoid(gi[:, :D] + gh[:, :D])
    z = _sigmoid(gi[:, D:2 * D] + gh[:, D:2 * D])
    n = jnp.tanh(gi[:, 2 * D:] + r * gh[:, 2 * D:])
    out_ref[...] = (1.0 - z) * n + z * h


def _gru(a0, a1, bc, h, wih_t, whh_t, bih, bhh):
    return pl.pallas_call(
        _gru_body,
        out_shape=jax.ShapeDtypeStruct((N, D), F32),
    )(a0, a1, bc, h, wih_t, whh_t, bih, bhh)


def _s2s_body(h_ref,
              w0a_ref, w0b_ref, u0_ref, b0_ref,
              w1_ref, u1_ref, b1_ref,
              w2_ref, u2_ref, b2_ref,
              wr1a_ref, wr1b_ref, br1_ref, wr2_ref, br2_ref,
              out_ref):
    h = h_ref[...]
    q = jnp.zeros((1, D), F32)
    rd = jnp.zeros((1, D), F32)
    hs = [jnp.zeros((1, D), F32) for _ in range(L)]
    cs = [jnp.zeros((1, D), F32) for _ in range(L)]
    ws = [(None, u0_ref, b0_ref), (w1_ref, u1_ref, b1_ref),
          (w2_ref, u2_ref, b2_ref)]
    hb = _b(h)
    for _ in range(S2S):
        x = None
        for l in range(L):
            w, u, b = ws[l]
            if l == 0:
                g = _dot(_b(q), w0a_ref[...]) + _dot(rd, w0b_ref[...])
            else:
                g = _dot(x, w[...])
            g = g + _dot(hs[l], u[...]) + b[...]
            i = _sigmoid(g[:, :D])
            f = _sigmoid(g[:, D:2 * D])
            gg = jnp.tanh(g[:, 2 * D:3 * D])
            o = _sigmoid(g[:, 3 * D:])
            cs[l] = f * cs[l] + i * gg
            hs[l] = o * jnp.tanh(cs[l])
            x = hs[l]
        q = x
        e = jnp.sum(hb * _b(q), axis=1, keepdims=True)
        e = e - jnp.max(e, axis=0, keepdims=True)
        ex = jnp.exp(e)
        alpha = ex / jnp.sum(ex, axis=0, keepdims=True)
        rd = jnp.sum(_b(alpha) * hb, axis=0, keepdims=True)
    hid = jnp.maximum(
        _dot(q, wr1a_ref[...]) + _dot(rd, wr1b_ref[...])
        + br1_ref[...], 0.0)
    out_ref[...] = _dot(hid, wr2_ref[...]) + br2_ref[...]


def _s2s(h, args):
    return pl.pallas_call(
        _s2s_body,
        out_shape=jax.ShapeDtypeStruct((1, NT), F32),
    )(h, *args)


@functools.cache
def _mesh():
    return plsc.VectorSubcoreMesh(core_axis_name="c", subcore_axis_name="s",
                                  num_cores=NC, num_subcores=NS)


def _sc_gather_body(h_hbm, src_hbm, out_hbm, idx_v, rows_v, sem):
    cid = lax.axis_index("c")
    sid = lax.axis_index("s")
    wid = sid * NC + cid
    cstart = wid * WCH
    pltpu.sync_copy(src_hbm.at[pl.ds(cstart, WCH)], idx_v)

    @pl.loop(0, NGRP)
    def _grp(g):
        descs = []
        for j in range(GRP):
            descs.append(pltpu.async_copy(
                h_hbm.at[idx_v.at[g * GRP + j, 0]], rows_v.at[j], sem))
        for d in descs:
            d.wait()
        pltpu.sync_copy(rows_v, out_hbm.at[pl.ds(cstart + g * GRP, GRP)])


@functools.cache
def _sc_gather_kernel():
    return pl.kernel(
        _sc_gather_body,
        out_type=jax.ShapeDtypeStruct((NCHUNK, CH, D), F32),
        mesh=_mesh(),
        scratch_types=[
            pltpu.VMEM((WCH, 1, CH), jnp.int32),
            pltpu.VMEM((GRP, CH, D), F32),
            pltpu.SemaphoreType.DMA,
        ],
        compiler_params=pltpu.CompilerParams(use_tc_tiling_on_sc=False),
    )


def _sc_gather(h, src3):
    return _sc_gather_kernel()(h, src3).reshape(E, D)


NEXP = 10


def _sc_scatter_body(m_hbm, dst_hbm, zero_hbm, out_hbm, shared, idx_v,
                     rows_v, sem):
    cid = lax.axis_index("c")
    sid = lax.axis_index("s")
    wid = sid * NC + cid
    cstart = wid * WCH

    @pl.when(sid == 0)
    def _zero():
        pltpu.sync_copy(zero_hbm, shared)

    plsc.subcore_barrier()
    pltpu.sync_copy(dst_hbm.at[pl.ds(cstart, WCH)], idx_v)

    @pl.loop(0, NGRP)
    def _grp(g):
        pltpu.sync_copy(m_hbm.at[pl.ds(cstart + g * GRP, GRP)], rows_v)
        for j in range(GRP):
            pltpu.sync_copy(rows_v.at[j],
                            shared.at[idx_v.at[g * GRP + j, 0]], add=True)

    plsc.subcore_barrier()

    @pl.when(sid < NEXP)
    def _export():
        rpt = N // NEXP
        pltpu.sync_copy(shared.at[pl.ds(sid * rpt, rpt)],
                        out_hbm.at[cid].at[pl.ds(sid * rpt, rpt)])


@functools.cache
def _sc_scatter_kernel():
    return pl.kernel(
        _sc_scatter_body,
        out_type=jax.ShapeDtypeStruct((NC, N, D), F32),
        mesh=_mesh(),
        scratch_types=[
            pltpu.VMEM_SHARED((N, D), F32),
            pltpu.VMEM((WCH, 1, CH), jnp.int32),
            pltpu.VMEM((GRP, CH, D), F32),
            pltpu.SemaphoreType.DMA,
        ],
        compiler_params=pltpu.CompilerParams(use_tc_tiling_on_sc=False),
    )


def _sc_scatter(m, dst3, zero):
    return _sc_scatter_kernel()(m.reshape(NCHUNK, CH, D), dst3, zero)


def kernel(node_feats, edge_index, edge_feats, W_p, b_p, We1, be1, We2, be2,
           b_conv, gru_Wih, gru_Whh, gru_bih, gru_bhh,
           lstm_Wih0, lstm_Whh0, lstm_bih0, lstm_bhh0,
           lstm_Wih1, lstm_Whh1, lstm_bih1, lstm_bhh1,
           lstm_Wih2, lstm_Whh2, lstm_bih2, lstm_bhh2,
           Wr1, br1, Wr2, br2):
    src3 = edge_index[0].astype(jnp.int32).reshape(NCHUNK, 1, CH)
    dst3 = edge_index[1].astype(jnp.int32).reshape(NCHUNK, 1, CH)

    ew = _edgenet(edge_feats, We1, be1.reshape(1, EH), We2,
                  be2.reshape(1, D * D), jnp.bfloat16)
    h = _proj(node_feats, W_p, b_p.reshape(1, D))

    wih_t = gru_Wih.T
    whh_t = gru_Whh.T
    bih = gru_bih.reshape(1, 3 * D)
    bhh = gru_bhh.reshape(1, 3 * D)
    bc = b_conv.reshape(1, D)
    zero = jnp.zeros((N, D), F32)
    bmat = jnp.kron(jnp.eye(D, dtype=jnp.bfloat16),
                    jnp.ones((1, D), jnp.bfloat16))
    rmat = jnp.tile(jnp.eye(D, dtype=F32), (D, 1))

    for _ in range(STEPS):
        hg = _sc_gather(h, src3)
        m = _msg(hg, ew, bmat, rmat)
        aggp = _sc_scatter(m, dst3, zero)
        h = _gru(aggp[0], aggp[1], bc, h, wih_t, whh_t, bih, bhh)

    w0t = lstm_Wih0.T
    s2s_args = (
        w0t[:D], w0t[D:], lstm_Whh0.T, lstm_bih0.reshape(1, 4 * D)
        + lstm_bhh0.reshape(1, 4 * D),
        lstm_Wih1.T, lstm_Whh1.T, lstm_bih1.reshape(1, 4 * D)
        + lstm_bhh1.reshape(1, 4 * D),
        lstm_Wih2.T, lstm_Whh2.T, lstm_bih2.reshape(1, 4 * D)
        + lstm_bhh2.reshape(1, 4 * D),
        Wr1[:D], Wr1[D:], br1.reshape(1, RH), Wr2, br2.reshape(1, NT),
    )
    return _s2s(h, s2s_args)
